# Initial kernel scaffold; baseline (speedup 1.0000x reference)
import jax
import jax.numpy as jnp
from jax.experimental import pallas as pl


def kernel(x, edge_index, W1, b1, W2, b2, W3, b3):
    raise NotImplementedError("write your pallas kernel here")


# speedup vs baseline: 16.9186x; 16.9186x over previous
import functools

import jax
import jax.numpy as jnp
from jax import lax
from jax.experimental import pallas as pl
from jax.experimental.pallas import tpu as pltpu
from jax.experimental.pallas import tpu_sc as plsc

N = 10000
D = 128
H1 = 512
H2 = 64

NC = 2
NS = 16
NW = NC * NS
CHUNK = 128
DW = 128
N_PAD = 10112
RPT = N_PAD // NS

_MESH = dict(core_axis_name="c", subcore_axis_name="s")


def _sc_degree(dstp, ones_rows, zeros_dw, *, G):
  E_pt = G * CHUNK

  @functools.partial(
      pl.kernel,
      out_type=jax.ShapeDtypeStruct((NC * N_PAD, DW), jnp.float32),
      mesh=plsc.VectorSubcoreMesh(**_MESH),
      scratch_types=[
          pltpu.VMEM((CHUNK,), jnp.int32),
          pltpu.VMEM((CHUNK, DW), jnp.float32),
          pltpu.VMEM_SHARED((N_PAD, DW), jnp.float32),
      ],
  )
  def k(dst_hbm, ones_hbm, zeros_hbm, out_hbm, idx_v, ones_v, deg_sh):
    c = lax.axis_index("c")
    s = lax.axis_index("s")
    pltpu.sync_copy(zeros_hbm.at[pl.ds(s * RPT, RPT)],
                    deg_sh.at[pl.ds(s * RPT, RPT)])
    pltpu.sync_copy(ones_hbm, ones_v)
    plsc.subcore_barrier()
    base = (s * NC + c) * E_pt

    def body(g, carry):
      off = base + g * CHUNK
      pltpu.sync_copy(dst_hbm.at[pl.ds(off, CHUNK)], idx_v)
      pltpu.sync_copy(ones_v, deg_sh.at[idx_v], add=True)
      return carry

    lax.fori_loop(0, G, body, 0)
    plsc.subcore_barrier()
    pltpu.sync_copy(deg_sh.at[pl.ds(s * RPT, RPT)],
                    out_hbm.at[pl.ds(c * N_PAD + s * RPT, RPT)])

  return k(dstp, ones_rows, zeros_dw)


def _sc_aggregate(y, srcp, dstp, zeros_d, *, G):
  E_pt = G * CHUNK

  @functools.partial(
      pl.kernel,
      out_type=jax.ShapeDtypeStruct((NC * N_PAD, D), jnp.float32),
      mesh=plsc.VectorSubcoreMesh(**_MESH),
      scratch_types=[
          pltpu.VMEM((CHUNK,), jnp.int32),
          pltpu.VMEM((CHUNK,), jnp.int32),
          pltpu.VMEM((CHUNK, D), jnp.float32),
          pltpu.SemaphoreType.DMA,
          pltpu.VMEM_SHARED((N_PAD, D), jnp.float32),
      ],
  )
  def k(y_hbm, src_hbm, dst_hbm, zeros_hbm, out_hbm,
        sidx_v, didx_v, rows_v, sem, z_sh):
    c = lax.axis_index("c")
    s = lax.axis_index("s")
    pltpu.sync_copy(zeros_hbm.at[pl.ds(s * RPT, RPT)],
                    z_sh.at[pl.ds(s * RPT, RPT)])
    plsc.subcore_barrier()
    base = (s * NC + c) * E_pt

    def body(g, carry):
      off = base + g * CHUNK
      pltpu.sync_copy(src_hbm.at[pl.ds(off, CHUNK)], sidx_v)
      pltpu.sync_copy(dst_hbm.at[pl.ds(off, CHUNK)], didx_v)
      pltpu.async_copy(y_hbm.at[sidx_v], rows_v, sem).wait()
      pltpu.sync_copy(rows_v, z_sh.at[didx_v], add=True)
      return carry

    lax.fori_loop(0, G, body, 0)
    plsc.subcore_barrier()
    pltpu.sync_copy(z_sh.at[pl.ds(s * RPT, RPT)],
                    out_hbm.at[pl.ds(c * N_PAD + s * RPT, RPT)])

  return k(y, srcp, dstp, zeros_d)


def _tc_scale(degp, x):
  def body(deg_ref, x_ref, y_ref, dinv_ref):
    dsum = deg_ref[0] + deg_ref[1]
    dinv = lax.rsqrt(dsum[:, 0:1] + 1.0)
    dinv_ref[...] = dinv
    y_ref[...] = x_ref[...] * dinv[:N]

  return pl.pallas_call(
      body,
      out_shape=(
          jax.ShapeDtypeStruct((N, D), jnp.float32),
          jax.ShapeDtypeStruct((N_PAD, 1), jnp.float32),
      ),
  )(degp.reshape(NC, N_PAD, DW), x)


def _tc_mlp(z, y, dinv, W1, b1, W2, b2, W3, b3):
  R = 1000
  zr = z.reshape(NC, N_PAD, D)

  def body(z0_ref, z1_ref, y_ref, dinv_ref, w1_ref, b1_ref, w2_ref, b2_ref,
           w3_ref, b3_ref, o_ref):
    u = (z0_ref[0] + z1_ref[0] + y_ref[...]) * dinv_ref[...]
    h = jnp.dot(u, w1_ref[...], preferred_element_type=jnp.float32)
    h = jnp.maximum(h + b1_ref[...], 0.0)
    h = jnp.dot(h, w2_ref[...], preferred_element_type=jnp.float32)
    h = jnp.maximum(h + b2_ref[...], 0.0)
    o = jnp.dot(h, w3_ref[...], preferred_element_type=jnp.float32)
    o_ref[...] = o + b3_ref[...]

  return pl.pallas_call(
      body,
      grid=(N // R,),
      in_specs=[
          pl.BlockSpec((1, R, D), lambda i: (0, i, 0)),
          pl.BlockSpec((1, R, D), lambda i: (1, i, 0)),
          pl.BlockSpec((R, D), lambda i: (i, 0)),
          pl.BlockSpec((R, 1), lambda i: (i, 0)),
          pl.BlockSpec((D, H1), lambda i: (0, 0)),
          pl.BlockSpec((1, H1), lambda i: (0, 0)),
          pl.BlockSpec((H1, H2), lambda i: (0, 0)),
          pl.BlockSpec((1, H2), lambda i: (0, 0)),
          pl.BlockSpec((H2, 1), lambda i: (0, 0)),
          pl.BlockSpec((1, 1), lambda i: (0, 0)),
      ],
      out_specs=pl.BlockSpec((R, 1), lambda i: (i, 0)),
      out_shape=jax.ShapeDtypeStruct((N, 1), jnp.float32),
  )(zr, zr, y, dinv, W1, b1.reshape(1, H1), W2, b2.reshape(1, H2),
    W3, b3.reshape(1, 1))


def kernel(x, edge_index, W1, b1, W2, b2, W3, b3):
  E = edge_index.shape[1]
  G = -(-E // (NW * CHUNK))
  E_pad = NW * G * CHUNK
  pad = E_pad - E
  src = edge_index[0]
  dst = edge_index[1]
  srcp = jnp.concatenate([src, jnp.zeros((pad,), jnp.int32)])
  dstp = jnp.concatenate([dst, jnp.full((pad,), N, jnp.int32)])
  ones_rows = jnp.ones((CHUNK, DW), jnp.float32)
  zeros_dw = jnp.zeros((N_PAD, DW), jnp.float32)
  zeros_d = jnp.zeros((N_PAD, D), jnp.float32)

  degp = _sc_degree(dstp, ones_rows, zeros_dw, G=G)
  y, dinv = _tc_scale(degp, x)
  z = _sc_aggregate(y, srcp, dstp, zeros_d, G=G)
  return _tc_mlp(z, y, dinv, W1, b1, W2, b2, W3, b3)


# speedup vs baseline: 19.1276x; 1.1306x over previous
import functools

import jax
import jax.numpy as jnp
from jax import lax
from jax.experimental import pallas as pl
from jax.experimental.pallas import tpu as pltpu
from jax.experimental.pallas import tpu_sc as plsc

N = 10000
D = 128
H1 = 512
H2 = 64

NC = 2
NS = 16
NW = NC * NS
CHUNK = 128
N_PAD = 10112
RPT = N_PAD // NS

_MESH = dict(core_axis_name="c", subcore_axis_name="s")


def _sc_degree(dstp, ones_rows, zeros_d, *, G):
  E_pt = G * CHUNK

  @functools.partial(
      pl.kernel,
      out_type=jax.ShapeDtypeStruct((NC * N_PAD, D), jnp.float32),
      mesh=plsc.VectorSubcoreMesh(**_MESH),
      scratch_types=[
          pltpu.VMEM((2, CHUNK), jnp.int32),
          pltpu.VMEM((CHUNK, D), jnp.float32),
          pltpu.SemaphoreType.DMA,
          pltpu.SemaphoreType.DMA,
          pltpu.VMEM_SHARED((N_PAD, D), jnp.float32),
      ],
  )
  def k(dst_hbm, ones_hbm, zeros_hbm, out_hbm,
        didx_v, ones_v, sem_a, sem_b, deg_sh):
    c = lax.axis_index("c")
    s = lax.axis_index("s")
    pltpu.sync_copy(zeros_hbm.at[pl.ds(s * RPT, RPT)],
                    deg_sh.at[pl.ds(s * RPT, RPT)])
    pltpu.sync_copy(ones_hbm, ones_v)
    plsc.subcore_barrier()
    base = (s * NC + c) * E_pt
    sem_i = (sem_a, sem_b)

    def idx_fire(g, b):
      pltpu.async_copy(dst_hbm.at[pl.ds(base + g * CHUNK, CHUNK)],
                       didx_v.at[b], sem_i[b])

    def idx_wait(b):
      pltpu.make_async_copy(dst_hbm.at[pl.ds(0, CHUNK)], didx_v.at[b],
                            sem_i[b]).wait()

    def scat(b):
      pltpu.sync_copy(ones_v, deg_sh.at[didx_v.at[b]], add=True)

    idx_fire(0, 0)

    def body(t, carry):
      g = 2 * t
      idx_wait(0)
      idx_fire(g + 1, 1)
      scat(0)
      idx_wait(1)
      idx_fire(g + 2, 0)
      scat(1)
      return carry

    lax.fori_loop(0, G // 2, body, 0)
    idx_wait(0)
    plsc.subcore_barrier()
    pltpu.sync_copy(deg_sh.at[pl.ds(s * RPT, RPT)],
                    out_hbm.at[pl.ds(c * N_PAD + s * RPT, RPT)])

  return k(dstp, ones_rows, zeros_d)


def _sc_aggregate(y, srcp, dstp, zeros_d, *, G):
  E_pt = G * CHUNK

  @functools.partial(
      pl.kernel,
      out_type=jax.ShapeDtypeStruct((NC * N_PAD, D), jnp.float32),
      mesh=plsc.VectorSubcoreMesh(**_MESH),
      scratch_types=[
          pltpu.VMEM((2, CHUNK), jnp.int32),
          pltpu.VMEM((2, CHUNK), jnp.int32),
          pltpu.VMEM((CHUNK, D), jnp.float32),
          pltpu.VMEM((CHUNK, D), jnp.float32),
          pltpu.SemaphoreType.DMA,
          pltpu.SemaphoreType.DMA,
          pltpu.SemaphoreType.DMA,
          pltpu.SemaphoreType.DMA,
          pltpu.VMEM_SHARED((N_PAD, D), jnp.float32),
      ],
  )
  def k(y_hbm, src_hbm, dst_hbm, zeros_hbm, out_hbm,
        sidx_v, didx_v, rows_a, rows_b, sem_ia, sem_ib, sem_ra, sem_rb, z_sh):
    c = lax.axis_index("c")
    s = lax.axis_index("s")
    pltpu.sync_copy(zeros_hbm.at[pl.ds(s * RPT, RPT)],
                    z_sh.at[pl.ds(s * RPT, RPT)])
    plsc.subcore_barrier()
    base = (s * NC + c) * E_pt
    rows = (rows_a, rows_b)
    sem_i = (sem_ia, sem_ib)
    sem_r = (sem_ra, sem_rb)

    def idx_fire(g, b):
      off = base + g * CHUNK
      pltpu.async_copy(src_hbm.at[pl.ds(off, CHUNK)], sidx_v.at[b], sem_i[b])
      pltpu.async_copy(dst_hbm.at[pl.ds(off, CHUNK)], didx_v.at[b], sem_i[b])

    def idx_wait(b):
      pltpu.make_async_copy(src_hbm.at[pl.ds(0, CHUNK)], sidx_v.at[b],
                            sem_i[b]).wait()
      pltpu.make_async_copy(src_hbm.at[pl.ds(0, CHUNK)], didx_v.at[b],
                            sem_i[b]).wait()

    def rows_fire(b):
      pltpu.async_copy(y_hbm.at[sidx_v.at[b]], rows[b], sem_r[b])

    def rows_wait(b):
      pltpu.make_async_copy(y_hbm.at[pl.ds(0, CHUNK)], rows[b],
                            sem_r[b]).wait()

    def scat(b):
      pltpu.sync_copy(rows[b], z_sh.at[didx_v.at[b]], add=True)

    idx_fire(0, 0)
    idx_wait(0)
    rows_fire(0)
    idx_fire(1, 1)

    def body(t, carry):
      g = 2 * t
      idx_wait(1)
      rows_wait(0)
      rows_fire(1)
      scat(0)
      idx_fire(g + 2, 0)
      idx_wait(0)
      rows_wait(1)
      rows_fire(0)
      scat(1)
      idx_fire(g + 3, 1)
      return carry

    lax.fori_loop(0, G // 2, body, 0)
    rows_wait(0)
    idx_wait(1)
    plsc.subcore_barrier()
    pltpu.sync_copy(z_sh.at[pl.ds(s * RPT, RPT)],
                    out_hbm.at[pl.ds(c * N_PAD + s * RPT, RPT)])

  return k(y, srcp, dstp, zeros_d)


def _tc_scale(degp, x):
  def body(deg_ref, x_ref, y_ref, dinv_ref):
    dsum = deg_ref[0] + deg_ref[1]
    dinv = lax.rsqrt(dsum[:, 0:1] + 1.0)
    dinv_ref[...] = dinv
    y_ref[...] = x_ref[...] * dinv[:N]

  return pl.pallas_call(
      body,
      out_shape=(
          jax.ShapeDtypeStruct((N, D), jnp.float32),
          jax.ShapeDtypeStruct((N_PAD, 1), jnp.float32),
      ),
  )(degp.reshape(NC, N_PAD, D), x)


def _tc_mlp(z, y, dinv, W1, b1, W2, b2, W3, b3):
  R = 1000
  zr = z.reshape(NC, N_PAD, D)

  def body(z0_ref, z1_ref, y_ref, dinv_ref, w1_ref, b1_ref, w2_ref, b2_ref,
           w3_ref, b3_ref, o_ref):
    u = (z0_ref[0] + z1_ref[0] + y_ref[...]) * dinv_ref[...]
    h = jnp.dot(u, w1_ref[...], preferred_element_type=jnp.float32)
    h = jnp.maximum(h + b1_ref[...], 0.0)
    h = jnp.dot(h, w2_ref[...], preferred_element_type=jnp.float32)
    h = jnp.maximum(h + b2_ref[...], 0.0)
    o = jnp.dot(h, w3_ref[...], preferred_element_type=jnp.float32)
    o_ref[...] = o + b3_ref[...]

  return pl.pallas_call(
      body,
      grid=(N // R,),
      in_specs=[
          pl.BlockSpec((1, R, D), lambda i: (0, i, 0)),
          pl.BlockSpec((1, R, D), lambda i: (1, i, 0)),
          pl.BlockSpec((R, D), lambda i: (i, 0)),
          pl.BlockSpec((R, 1), lambda i: (i, 0)),
          pl.BlockSpec((D, H1), lambda i: (0, 0)),
          pl.BlockSpec((1, H1), lambda i: (0, 0)),
          pl.BlockSpec((H1, H2), lambda i: (0, 0)),
          pl.BlockSpec((1, H2), lambda i: (0, 0)),
          pl.BlockSpec((H2, 1), lambda i: (0, 0)),
          pl.BlockSpec((1, 1), lambda i: (0, 0)),
      ],
      out_specs=pl.BlockSpec((R, 1), lambda i: (i, 0)),
      out_shape=jax.ShapeDtypeStruct((N, 1), jnp.float32),
  )(zr, zr, y, dinv, W1, b1.reshape(1, H1), W2, b2.reshape(1, H2),
    W3, b3.reshape(1, 1))


def kernel(x, edge_index, W1, b1, W2, b2, W3, b3):
  E = edge_index.shape[1]
  G = 2 * (-(-E // (NW * CHUNK * 2)))
  E_pad = NW * G * CHUNK
  pad = E_pad - E + 2 * CHUNK
  src = edge_index[0]
  dst = edge_index[1]
  srcp = jnp.concatenate([src, jnp.zeros((pad,), jnp.int32)])
  dstp = jnp.concatenate([dst, jnp.full((pad,), N, jnp.int32)])
  ones_rows = jnp.ones((CHUNK, D), jnp.float32)
  zeros_d = jnp.zeros((N_PAD, D), jnp.float32)

  degp = _sc_degree(dstp, ones_rows, zeros_d, G=G)
  y, dinv = _tc_scale(degp, x)
  z = _sc_aggregate(y, srcp, dstp, zeros_d, G=G)
  return _tc_mlp(z, y, dinv, W1, b1, W2, b2, W3, b3)
